# baseline (speedup 1.0000x reference)
import functools

import jax
import jax.numpy as jnp
from jax import lax
from jax.experimental import pallas as pl
from jax.experimental.pallas import tpu as pltpu
from jax.experimental.pallas import tpu_sc as plsc

_NC, _NS = 2, 16
_NW = _NC * _NS
_GSZ = 128


@functools.lru_cache(maxsize=None)
def _make_gather(n_rows: int, feat: int):
    assert n_rows % (_NW * _GSZ) == 0
    n_groups = n_rows // _GSZ
    groups_per_w = n_groups // _NW
    mesh = plsc.VectorSubcoreMesh(core_axis_name="c", subcore_axis_name="s")

    K = 2
    S = groups_per_w // K
    assert groups_per_w % K == 0 and S % 2 == 0

    @functools.partial(
        pl.kernel,
        out_type=jax.ShapeDtypeStruct((n_rows, feat), jnp.float32),
        mesh=mesh,
        scratch_types=[
            pltpu.VMEM((groups_per_w, _GSZ), jnp.int32),
            pltpu.VMEM((K * _GSZ, feat), jnp.float32),
            pltpu.VMEM((K * _GSZ, feat), jnp.float32),
            pltpu.SemaphoreType.DMA,
            pltpu.SemaphoreType.DMA,
            pltpu.SemaphoreType.DMA,
        ],
        compiler_params=pltpu.CompilerParams(use_tc_tiling_on_sc=False),
    )
    def gather_kernel(table2_hbm, idx_hbm, out_hbm, idx_v, rows_a, rows_b,
                      gsem, wsem_a, wsem_b):
        table_hbm = table2_hbm
        wid = lax.axis_index("s") * _NC + lax.axis_index("c")
        base_g = wid * groups_per_w
        pltpu.sync_copy(idx_hbm.at[pl.ds(base_g, groups_per_w)], idx_v)

        def fire_gathers(buf, ss):
            for t in range(K):
                pltpu.async_copy(table_hbm.at[idx_v.at[ss * K + t]],
                                 buf.at[pl.ds(t * _GSZ, _GSZ)], gsem)

        def out_slice(ss):
            return out_hbm.at[pl.ds((base_g + ss * K) * _GSZ, K * _GSZ)]

        fire_gathers(rows_a, 0)

        @pl.loop(0, S, step=2)
        def _(s):
            for cur, nxt, wsem_cur, wsem_nxt, off in (
                    (rows_a, rows_b, wsem_a, wsem_b, 0),
                    (rows_b, rows_a, wsem_b, wsem_a, 1)):
                ss = s + off
                for t in range(K):
                    pltpu.make_async_copy(
                        table_hbm.at[idx_v.at[ss * K + t]],
                        cur.at[pl.ds(t * _GSZ, _GSZ)], gsem).wait()
                @pl.when(ss + 1 < S)
                def _():
                    @pl.when(ss >= 1)
                    def _():
                        pltpu.make_async_copy(nxt, out_slice(ss - 1),
                                              wsem_nxt).wait()
                    fire_gathers(nxt, ss + 1)
                pltpu.async_copy(cur, out_slice(ss), wsem_cur)

        pltpu.make_async_copy(rows_a, out_slice(S - 2), wsem_a).wait()
        pltpu.make_async_copy(rows_b, out_slice(S - 1), wsem_b).wait()

    return gather_kernel


def kernel(image, W):
    B, A = image.shape
    V, F = W.shape
    idx = image.reshape(-1).astype(jnp.int32)
    n_rows = B * A
    idx2 = idx.reshape(n_rows // _GSZ, _GSZ)
    Wp = jnp.pad(W, ((0, 0), (0, 2 * F - ((-F) % 128) - F + ((-F) % 128))))
    Wp = jnp.pad(W, ((0, 0), (0, 128 - F)))
    rows = _make_gather(n_rows, 128)(Wp, idx2)
    return rows[:, :F].reshape(B, A * F)


# speedup vs baseline: 1.1100x; 1.1100x over previous
import functools

import jax
import jax.numpy as jnp
from jax import lax
from jax.experimental import pallas as pl
from jax.experimental.pallas import tpu as pltpu
from jax.experimental.pallas import tpu_sc as plsc

_NC, _NS = 2, 16
_NW = _NC * _NS
_GSZ = 128


@functools.lru_cache(maxsize=None)
def _make_gather(n_rows: int, feat: int):
    assert n_rows % (_NW * _GSZ) == 0
    n_groups = n_rows // _GSZ
    groups_per_w = n_groups // _NW
    mesh = plsc.VectorSubcoreMesh(core_axis_name="c", subcore_axis_name="s")

    K = 4
    S = groups_per_w // K
    assert groups_per_w % K == 0 and S % 2 == 0

    @functools.partial(
        pl.kernel,
        out_type=jax.ShapeDtypeStruct((n_rows, feat), jnp.float32),
        mesh=mesh,
        scratch_types=[
            pltpu.VMEM((groups_per_w, _GSZ), jnp.int32),
            pltpu.VMEM((K * _GSZ, feat), jnp.float32),
            pltpu.VMEM((K * _GSZ, feat), jnp.float32),
            pltpu.SemaphoreType.DMA,
            pltpu.SemaphoreType.DMA,
            pltpu.SemaphoreType.DMA,
        ],
        compiler_params=pltpu.CompilerParams(use_tc_tiling_on_sc=False),
    )
    def gather_kernel(table_hbm, idx_hbm, out_hbm, idx_v, rows_a, rows_b,
                      gsem, wsem_a, wsem_b):
        wid = lax.axis_index("s") * _NC + lax.axis_index("c")
        base_g = wid * groups_per_w
        pltpu.sync_copy(idx_hbm.at[pl.ds(base_g, groups_per_w)], idx_v)

        def fire_gathers(buf, ss):
            for t in range(K):
                pltpu.async_copy(table_hbm.at[idx_v.at[ss * K + t]],
                                 buf.at[pl.ds(t * _GSZ, _GSZ)], gsem)

        def out_slice(ss):
            return out_hbm.at[pl.ds((base_g + ss * K) * _GSZ, K * _GSZ)]

        fire_gathers(rows_a, 0)

        @pl.loop(0, S, step=2)
        def _(s):
            for cur, nxt, wsem_cur, wsem_nxt, off in (
                    (rows_a, rows_b, wsem_a, wsem_b, 0),
                    (rows_b, rows_a, wsem_b, wsem_a, 1)):
                ss = s + off
                for t in range(K):
                    pltpu.make_async_copy(
                        table_hbm.at[idx_v.at[ss * K + t]],
                        cur.at[pl.ds(t * _GSZ, _GSZ)], gsem).wait()
                @pl.when(ss + 1 < S)
                def _():
                    @pl.when(ss >= 1)
                    def _():
                        pltpu.make_async_copy(nxt, out_slice(ss - 1),
                                              wsem_nxt).wait()
                    fire_gathers(nxt, ss + 1)
                pltpu.async_copy(cur, out_slice(ss), wsem_cur)

        pltpu.make_async_copy(rows_a, out_slice(S - 2), wsem_a).wait()
        pltpu.make_async_copy(rows_b, out_slice(S - 1), wsem_b).wait()

    return gather_kernel


def kernel(image, W):
    B, A = image.shape
    V, F = W.shape
    n_rows = B * A
    TB, TA = B // 8, (A * F) // 128
    PAIR = 128 // F
    idxp = (image.astype(jnp.int32)
            .reshape(TB, 8, TA, PAIR)
            .transpose(0, 2, 1, 3)
            .reshape(n_rows // _GSZ, _GSZ))
    rows = _make_gather(n_rows, F)(W, idxp)
    out = (rows.reshape(TB, TA, 8, 128)
           .transpose(0, 2, 1, 3)
           .reshape(B, A * F))
    return out


# speedup vs baseline: 1.2462x; 1.1228x over previous
import functools

import jax
import jax.numpy as jnp
from jax import lax
from jax.experimental import pallas as pl
from jax.experimental.pallas import tpu as pltpu
from jax.experimental.pallas import tpu_sc as plsc

_NC, _NS = 2, 16
_NW = _NC * _NS
_GSZ = 128
_L = 16


@functools.lru_cache(maxsize=None)
def _make_gather(n_rows: int, feat: int):
    assert n_rows % (_NW * _GSZ) == 0
    n_groups = n_rows // _GSZ
    groups_per_w = n_groups // _NW
    rows_per_w = groups_per_w * _GSZ
    mesh = plsc.VectorSubcoreMesh(core_axis_name="c", subcore_axis_name="s")

    K = 4
    S = groups_per_w // K
    assert groups_per_w % K == 0 and S % 2 == 0

    @functools.partial(
        pl.kernel,
        out_type=jax.ShapeDtypeStruct((n_rows, feat), jnp.float32),
        mesh=mesh,
        scratch_types=[
            pltpu.VMEM((rows_per_w,), jnp.int32),
            pltpu.VMEM((rows_per_w,), jnp.int32),
            pltpu.VMEM((rows_per_w,), jnp.int32),
            pltpu.VMEM((K * _GSZ, feat), jnp.float32),
            pltpu.VMEM((K * _GSZ, feat), jnp.float32),
            pltpu.SemaphoreType.DMA,
            pltpu.SemaphoreType.DMA,
            pltpu.SemaphoreType.DMA,
        ],
        compiler_params=pltpu.CompilerParams(use_tc_tiling_on_sc=False,
                                             needs_layout_passes=False),
    )
    def gather_kernel(table_hbm, idx_hbm, perm_hbm, out_hbm,
                      idx_raw, perm_v, idx_v, rows_a, rows_b,
                      gsem, wsem_a, wsem_b):
        wid = lax.axis_index("s") * _NC + lax.axis_index("c")
        base = wid * rows_per_w
        pltpu.sync_copy(idx_hbm.at[pl.ds(base, rows_per_w)], idx_raw)
        pltpu.sync_copy(perm_hbm, perm_v)

        @pl.loop(0, rows_per_w // _L, unroll=4)
        def _(t):
            pv = perm_v[pl.ds(t * _L, _L)]
            idx_v[pl.ds(t * _L, _L)] = plsc.load_gather(idx_raw, [pv])

        def fire_gathers(buf, ss):
            for t in range(K):
                pltpu.async_copy(
                    table_hbm.at[idx_v.at[pl.ds((ss * K + t) * _GSZ, _GSZ)]],
                    buf.at[pl.ds(t * _GSZ, _GSZ)], gsem)

        def out_slice(ss):
            return out_hbm.at[pl.ds(base + ss * K * _GSZ, K * _GSZ)]

        fire_gathers(rows_a, 0)

        @pl.loop(0, S, step=2)
        def _(s):
            for cur, nxt, wsem_cur, wsem_nxt, off in (
                    (rows_a, rows_b, wsem_a, wsem_b, 0),
                    (rows_b, rows_a, wsem_b, wsem_a, 1)):
                ss = s + off
                for t in range(K):
                    pltpu.make_async_copy(
                        table_hbm.at[idx_v.at[pl.ds((ss * K + t) * _GSZ,
                                                    _GSZ)]],
                        cur.at[pl.ds(t * _GSZ, _GSZ)], gsem).wait()
                @pl.when(ss + 1 < S)
                def _():
                    @pl.when(ss >= 1)
                    def _():
                        pltpu.make_async_copy(nxt, out_slice(ss - 1),
                                              wsem_nxt).wait()
                    fire_gathers(nxt, ss + 1)
                pltpu.async_copy(cur, out_slice(ss), wsem_cur)

        pltpu.make_async_copy(rows_a, out_slice(S - 2), wsem_a).wait()
        pltpu.make_async_copy(rows_b, out_slice(S - 1), wsem_b).wait()

    return gather_kernel


def kernel(image, W):
    B, A = image.shape
    V, F = W.shape
    n_rows = B * A
    idx = image.reshape(-1).astype(jnp.int32)
    TA = (A * F) // 128
    PAIR = 128 // F
    rows_per_w = n_rows // _NW
    IB = rows_per_w // (8 * A)
    perm = (jnp.arange(rows_per_w, dtype=jnp.int32)
            .reshape(IB, 8, TA, PAIR)
            .transpose(0, 2, 1, 3)
            .reshape(-1))
    rows = _make_gather(n_rows, F)(W, idx, perm)
    TB = B // 8
    out = (rows.reshape(TB, TA, 8, 128)
           .transpose(0, 2, 1, 3)
           .reshape(B, A * F))
    return out
